# baseline (speedup 1.0000x reference)
import functools

import jax
import jax.numpy as jnp
from jax import lax
from jax.experimental import pallas as pl
from jax.experimental.pallas import tpu as pltpu
from jax.experimental.pallas import tpu_sc as plsc

BATCH = 4096
VOCAB_ = 1000000
NUM_WORDS = 50
EMB = 32
LAT = 64
HALF = 16

NC = 2
NS = 16
NW = NC * NS
B_PER_W = BATCH // NW
NIDX = NUM_WORDS * B_PER_W

VP = 1000064
VMAIN = 999936

_mesh = plsc.VectorSubcoreMesh(core_axis_name="c", subcore_axis_name="s")


def _detile_body(tT_ref, tailp_ref, out_hbm, sems):
    j = pl.program_id(0)

    def main_copy(jj):
        return (tT_ref.at[jj, pl.ds(0, VMAIN)],
                out_hbm.at[pl.ds(jj * VP, VMAIN)])

    def tail_copy(jj):
        return (tailp_ref.at[pl.ds(jj * 128, 128)],
                out_hbm.at[pl.ds(jj * VP + VMAIN, 128)])

    @pl.when(j > 0)
    def _drain_prev():
        s, d = main_copy(j - 1)
        pltpu.make_async_copy(s, d, sems.at[(j - 1) % 2]).wait()
        s, d = tail_copy(j - 1)
        pltpu.make_async_copy(s, d, sems.at[(j - 1) % 2]).wait()

    s, d = main_copy(j)
    pltpu.async_copy(s, d, sems.at[j % 2])
    s, d = tail_copy(j)
    pltpu.async_copy(s, d, sems.at[j % 2])

    @pl.when(j == EMB - 1)
    def _drain_last():
        s, d = main_copy(j)
        pltpu.make_async_copy(s, d, sems.at[j % 2]).wait()
        s, d = tail_copy(j)
        pltpu.make_async_copy(s, d, sems.at[j % 2]).wait()


_detile = pl.pallas_call(
    _detile_body,
    grid=(EMB,),
    in_specs=[pl.BlockSpec(memory_space=pl.ANY),
              pl.BlockSpec(memory_space=pl.ANY)],
    out_specs=pl.BlockSpec(memory_space=pl.ANY),
    out_shape=jax.ShapeDtypeStruct((EMB * VP,), jnp.float32),
    scratch_shapes=[pltpu.SemaphoreType.DMA((2,))],
)


@functools.partial(
    pl.kernel,
    mesh=_mesh,
    compiler_params=pltpu.CompilerParams(use_tc_tiling_on_sc=False),
    out_type=jax.ShapeDtypeStruct((EMB, BATCH), jnp.float32),
    scratch_types=[
        pltpu.VMEM((NIDX,), jnp.int32),
        pltpu.VMEM((NIDX,), jnp.int32),
        pltpu.VMEM((NIDX,), jnp.float32),
        pltpu.VMEM((NIDX,), jnp.float32),
        pltpu.VMEM((EMB, B_PER_W), jnp.float32),
        pltpu.SemaphoreType.DMA,
        pltpu.SemaphoreType.DMA,
        pltpu.SemaphoreType.DMA,
    ],
)
def _sc_colpool(wT_hbm, tcol_hbm, outT_hbm, offs_a, offs_b, buf0, buf1,
                accT_v, sem0, sem1, sem_fill):
    wid = lax.axis_index("s") * NC + lax.axis_index("c")
    obase = wid * B_PER_W

    fills = []
    for k in range(NUM_WORDS):
        fills.append(pltpu.async_copy(
            wT_hbm.at[k, pl.ds(obase, B_PER_W)],
            offs_a.at[pl.ds(k * B_PER_W, B_PER_W)],
            sem_fill))
    for f in fills:
        f.wait()

    offs = (offs_a, offs_b)
    bufs = (buf0, buf1)
    sems = (sem0, sem1)
    handles = [None, None]
    scale = jnp.float32(1.0 / NUM_WORDS)

    def start(j):
        handles[j % 2] = pltpu.async_copy(
            tcol_hbm.at[offs[j % 2]], bufs[j % 2], sems[j % 2])

    def shift(j):
        def body(v, carry):
            sl = pl.ds(v * HALF, HALF)
            offs[(j + 1) % 2][sl] = offs[j % 2][sl] + VP
            return carry

        lax.fori_loop(0, NIDX // HALF, body, 0)

    def pool(j):
        buf = bufs[j % 2]

        def rb_body(rb, carry):
            base = rb * HALF
            a = [None] * 4
            for k in range(NUM_WORDS):
                v = buf[pl.ds(k * B_PER_W + base, HALF)]
                g = k % 4
                a[g] = v if a[g] is None else a[g] + v
            s = ((a[0] + a[1]) + (a[2] + a[3])) * scale
            accT_v[j, pl.ds(base, HALF)] = s
            return carry

        lax.fori_loop(0, B_PER_W // HALF, rb_body, 0)

    start(0)
    for j in range(1, EMB):
        shift(j - 1)
        start(j)
        handles[(j - 1) % 2].wait()
        pool(j - 1)
    handles[(EMB - 1) % 2].wait()
    pool(EMB - 1)

    pltpu.sync_copy(accT_v, outT_hbm.at[:, pl.ds(obase, B_PER_W)])


def _headsT_body(pT_ref, wmu_ref, bmu_ref, wsig_ref, bsig_ref,
                 mT_ref, lT_ref):
    pT = pT_ref[...]
    mT_ref[...] = lax.dot_general(
        wmu_ref[...], pT, (((0,), (0,)), ((), ())),
        preferred_element_type=jnp.float32) + bmu_ref[...]
    lT_ref[...] = lax.dot_general(
        wsig_ref[...], pT, (((0,), (0,)), ((), ())),
        preferred_element_type=jnp.float32) + bsig_ref[...]


_headsT = pl.pallas_call(
    _headsT_body,
    out_shape=(
        jax.ShapeDtypeStruct((LAT, BATCH), jnp.float32),
        jax.ShapeDtypeStruct((LAT, BATCH), jnp.float32),
    ),
)


def kernel(w, table, W_mu, b_mu, W_sig, b_sig):
    tailp = jnp.pad(table[VOCAB_ - 64:].T, ((0, 0), (0, 64))).reshape(-1)
    tcol = _detile(table.T, tailp)
    pooledT = _sc_colpool(w.T.astype(jnp.int32), tcol)
    meanT, logstdT = _headsT(
        pooledT, W_mu, b_mu.reshape(LAT, 1), W_sig, b_sig.reshape(LAT, 1))
    return (meanT.T, logstdT.T)


# speedup vs baseline: 1.4927x; 1.4927x over previous
import functools

import jax
import jax.numpy as jnp
from jax import lax
from jax.experimental import pallas as pl
from jax.experimental.pallas import tpu as pltpu
from jax.experimental.pallas import tpu_sc as plsc

BATCH = 4096
VOCAB_ = 1000000
NUM_WORDS = 50
EMB = 32
LAT = 64
HALF = 16

NC = 2
NS = 16
NW = NC * NS
B_PER_W = BATCH // NW
NIDX = NUM_WORDS * B_PER_W

VP = VOCAB_

_mesh = plsc.VectorSubcoreMesh(core_axis_name="c", subcore_axis_name="s")


@functools.partial(
    pl.kernel,
    mesh=_mesh,
    compiler_params=pltpu.CompilerParams(use_tc_tiling_on_sc=False),
    out_type=jax.ShapeDtypeStruct((EMB, BATCH), jnp.float32),
    scratch_types=[
        pltpu.VMEM((NIDX,), jnp.int32),
        pltpu.VMEM((NIDX,), jnp.int32),
        pltpu.VMEM((NIDX,), jnp.float32),
        pltpu.VMEM((NIDX,), jnp.float32),
        pltpu.VMEM((EMB, B_PER_W), jnp.float32),
        pltpu.SemaphoreType.DMA,
        pltpu.SemaphoreType.DMA,
        pltpu.SemaphoreType.DMA,
    ],
)
def _sc_colpool(wT_hbm, tcol_hbm, outT_hbm, offs_a, offs_b, buf0, buf1,
                accT_v, sem0, sem1, sem_fill):
    wid = lax.axis_index("s") * NC + lax.axis_index("c")
    obase = wid * B_PER_W

    fills = []
    for k in range(NUM_WORDS):
        fills.append(pltpu.async_copy(
            wT_hbm.at[k, pl.ds(obase, B_PER_W)],
            offs_a.at[pl.ds(k * B_PER_W, B_PER_W)],
            sem_fill))
    for f in fills:
        f.wait()

    offs = (offs_a, offs_b)
    bufs = (buf0, buf1)
    sems = (sem0, sem1)
    handles = [None, None]
    scale = jnp.float32(1.0 / NUM_WORDS)

    def start(j):
        handles[j % 2] = pltpu.async_copy(
            tcol_hbm.at[offs[j % 2]], bufs[j % 2], sems[j % 2])

    def shift(j):
        def body(v, carry):
            sl = pl.ds(v * HALF, HALF)
            offs[(j + 1) % 2][sl] = offs[j % 2][sl] + VP
            return carry

        lax.fori_loop(0, NIDX // HALF, body, 0)

    def pool(j):
        buf = bufs[j % 2]

        def rb_body(rb, carry):
            base = rb * HALF
            a = [None] * 4
            for k in range(NUM_WORDS):
                v = buf[pl.ds(k * B_PER_W + base, HALF)]
                g = k % 4
                a[g] = v if a[g] is None else a[g] + v
            s = ((a[0] + a[1]) + (a[2] + a[3])) * scale
            accT_v[j, pl.ds(base, HALF)] = s
            return carry

        lax.fori_loop(0, B_PER_W // HALF, rb_body, 0)

    start(0)
    for j in range(1, EMB):
        shift(j - 1)
        start(j)
        handles[(j - 1) % 2].wait()
        pool(j - 1)
    handles[(EMB - 1) % 2].wait()
    pool(EMB - 1)

    pltpu.sync_copy(accT_v, outT_hbm.at[:, pl.ds(obase, B_PER_W)])


def _headsT_body(pT_ref, wmu_ref, bmu_ref, wsig_ref, bsig_ref,
                 mT_ref, lT_ref):
    pT = pT_ref[...]
    mT_ref[...] = lax.dot_general(
        wmu_ref[...], pT, (((0,), (0,)), ((), ())),
        preferred_element_type=jnp.float32) + bmu_ref[...]
    lT_ref[...] = lax.dot_general(
        wsig_ref[...], pT, (((0,), (0,)), ((), ())),
        preferred_element_type=jnp.float32) + bsig_ref[...]


_headsT = pl.pallas_call(
    _headsT_body,
    out_shape=(
        jax.ShapeDtypeStruct((LAT, BATCH), jnp.float32),
        jax.ShapeDtypeStruct((LAT, BATCH), jnp.float32),
    ),
)


def kernel(w, table, W_mu, b_mu, W_sig, b_sig):
    tcol = jnp.reshape(table.T, (EMB * VP,))
    pooledT = _sc_colpool(w.T.astype(jnp.int32), tcol)
    meanT, logstdT = _headsT(
        pooledT, W_mu, b_mu.reshape(LAT, 1), W_sig, b_sig.reshape(LAT, 1))
    return (meanT.T, logstdT.T)


# speedup vs baseline: 7.8974x; 5.2907x over previous
import functools

import jax
import jax.numpy as jnp
from jax import lax
from jax.experimental import pallas as pl
from jax.experimental.pallas import tpu as pltpu
from jax.experimental.pallas import tpu_sc as plsc

BATCH = 4096
VOCAB_ = 1000000
NUM_WORDS = 50
EMB = 32
LAT = 64
HALF = 16

NC = 2
NS = 16
NW = NC * NS
B_PER_W = BATCH // NW
CHUNK = 16
NCHUNK = B_PER_W // CHUNK
ROWS = CHUNK * NUM_WORDS

_mesh = plsc.VectorSubcoreMesh(core_axis_name="c", subcore_axis_name="s")


KG = 10
NKG = NUM_WORDS // KG


@functools.partial(
    pl.kernel,
    mesh=_mesh,
    compiler_params=pltpu.CompilerParams(use_tc_tiling_on_sc=False),
    out_type=jax.ShapeDtypeStruct((BATCH, EMB), jnp.float32),
    scratch_types=[
        pltpu.VMEM((NUM_WORDS * B_PER_W,), jnp.int32),
        pltpu.VMEM((KG * B_PER_W, EMB), jnp.float32),
        pltpu.VMEM((KG * B_PER_W, EMB), jnp.float32),
        pltpu.VMEM((B_PER_W, EMB), jnp.float32),
        pltpu.SemaphoreType.DMA,
        pltpu.SemaphoreType.DMA,
        pltpu.SemaphoreType.DMA,
    ],
)
def _sc_pool(wT_hbm, table_hbm, out_hbm, idxk_v, buf0, buf1, acc_v,
             sem0, sem1, sem_fill):
    wid = lax.axis_index("s") * NC + lax.axis_index("c")
    obase = wid * B_PER_W

    fills = []
    for k in range(NUM_WORDS):
        fills.append(pltpu.async_copy(
            wT_hbm.at[k, pl.ds(obase, B_PER_W)],
            idxk_v.at[pl.ds(k * B_PER_W, B_PER_W)],
            sem_fill))
    for f in fills:
        f.wait()

    bufs = (buf0, buf1)
    sems = (sem0, sem1)
    handles = [None, None]

    def start(g):
        idx_sl = idxk_v.at[pl.ds(g * KG * B_PER_W, KG * B_PER_W)]
        handles[g % 2] = pltpu.async_copy(
            table_hbm.at[idx_sl], bufs[g % 2], sems[g % 2])

    def process(g):
        buf = bufs[g % 2]
        first = g == 0

        def row_body(r, carry):
            for h in range(2):
                sl = pl.ds(h * HALF, HALF)
                b = [buf[j * B_PER_W + r, sl] for j in range(KG)]
                s = (((b[0] + b[1]) + (b[2] + b[3]))
                     + ((b[4] + b[5]) + (b[6] + b[7]))) + (b[8] + b[9])
                if first:
                    acc_v[r, sl] = s
                else:
                    acc_v[r, sl] = acc_v[r, sl] + s
            return carry

        lax.fori_loop(0, B_PER_W, row_body, 0)

    start(0)
    for g in range(1, NKG):
        start(g)
        handles[(g - 1) % 2].wait()
        process(g - 1)
    handles[(NKG - 1) % 2].wait()
    process(NKG - 1)

    scale = jnp.float32(1.0 / NUM_WORDS)

    def scale_body(r, carry):
        for h in range(2):
            sl = pl.ds(h * HALF, HALF)
            acc_v[r, sl] = acc_v[r, sl] * scale
        return carry

    lax.fori_loop(0, B_PER_W, scale_body, 0)
    pltpu.sync_copy(acc_v, out_hbm.at[pl.ds(obase, B_PER_W)])


def _headsT_body(p_ref, wmu_ref, bmu_ref, wsig_ref, bsig_ref,
                 mT_ref, lT_ref):
    pooled = p_ref[...]
    mT_ref[...] = lax.dot_general(
        wmu_ref[...], pooled, (((0,), (1,)), ((), ())),
        preferred_element_type=jnp.float32) + bmu_ref[...]
    lT_ref[...] = lax.dot_general(
        wsig_ref[...], pooled, (((0,), (1,)), ((), ())),
        preferred_element_type=jnp.float32) + bsig_ref[...]


_headsT = pl.pallas_call(
    _headsT_body,
    out_shape=(
        jax.ShapeDtypeStruct((LAT, BATCH), jnp.float32),
        jax.ShapeDtypeStruct((LAT, BATCH), jnp.float32),
    ),
)


def kernel(w, table, W_mu, b_mu, W_sig, b_sig):
    pooled = _sc_pool(w.T.astype(jnp.int32), table)
    meanT, logstdT = _headsT(
        pooled, W_mu, b_mu.reshape(LAT, 1), W_sig, b_sig.reshape(LAT, 1))
    return (meanT.T, logstdT.T)


# speedup vs baseline: 9.2694x; 1.1737x over previous
import functools

import jax
import jax.numpy as jnp
from jax import lax
from jax.experimental import pallas as pl
from jax.experimental.pallas import tpu as pltpu
from jax.experimental.pallas import tpu_sc as plsc

BATCH = 4096
VOCAB_ = 1000000
NUM_WORDS = 50
EMB = 32
LAT = 64
HALF = 16

NC = 2
NS = 16
NW = NC * NS
B_PER_W = BATCH // NW
CHUNK = 16
NCHUNK = B_PER_W // CHUNK
ROWS = CHUNK * NUM_WORDS

_mesh = plsc.VectorSubcoreMesh(core_axis_name="c", subcore_axis_name="s")


KG = 10
NKG = NUM_WORDS // KG


@functools.partial(
    pl.kernel,
    mesh=_mesh,
    compiler_params=pltpu.CompilerParams(use_tc_tiling_on_sc=False),
    out_type=jax.ShapeDtypeStruct((BATCH, EMB), jnp.float32),
    scratch_types=[
        pltpu.VMEM((NUM_WORDS * B_PER_W,), jnp.int32),
        pltpu.VMEM((KG * B_PER_W, EMB), jnp.float32),
        pltpu.VMEM((KG * B_PER_W, EMB), jnp.float32),
        pltpu.VMEM((B_PER_W, EMB), jnp.float32),
        pltpu.SemaphoreType.DMA,
        pltpu.SemaphoreType.DMA,
        pltpu.SemaphoreType.DMA,
    ],
)
def _sc_pool(wT_hbm, table_hbm, out_hbm, idxk_v, buf0, buf1, acc_v,
             sem0, sem1, sem_fill):
    wid = lax.axis_index("s") * NC + lax.axis_index("c")
    obase = wid * B_PER_W

    fills = []
    for k in range(NUM_WORDS):
        fills.append(pltpu.async_copy(
            wT_hbm.at[k, pl.ds(obase, B_PER_W)],
            idxk_v.at[pl.ds(k * B_PER_W, B_PER_W)],
            sem_fill))
    for f in fills:
        f.wait()

    def xform_body(k, carry):
        sl = pl.ds(k * HALF, HALF)
        i = idxk_v[sl]
        v = jnp.bitwise_and(i, 2047)
        idxk_v[sl] = ((i - v)
                      + jnp.left_shift(jnp.bitwise_and(v, 511), 2)
                      + jnp.right_shift(v, 9))
        return carry

    lax.fori_loop(0, NUM_WORDS * B_PER_W // HALF, xform_body, 0)

    bufs = (buf0, buf1)
    sems = (sem0, sem1)
    handles = [None, None]

    def start(g):
        idx_sl = idxk_v.at[pl.ds(g * KG * B_PER_W, KG * B_PER_W)]
        handles[g % 2] = pltpu.async_copy(
            table_hbm.at[idx_sl], bufs[g % 2], sems[g % 2])

    def process(g):
        buf = bufs[g % 2]
        first = g == 0

        def row_body(r, carry):
            for h in range(2):
                sl = pl.ds(h * HALF, HALF)
                b = [buf[j * B_PER_W + r, sl] for j in range(KG)]
                s = (((b[0] + b[1]) + (b[2] + b[3]))
                     + ((b[4] + b[5]) + (b[6] + b[7]))) + (b[8] + b[9])
                if first:
                    acc_v[r, sl] = s
                else:
                    acc_v[r, sl] = acc_v[r, sl] + s
            return carry

        lax.fori_loop(0, B_PER_W, row_body, 0)

    start(0)
    for g in range(1, NKG):
        start(g)
        handles[(g - 1) % 2].wait()
        process(g - 1)
    handles[(NKG - 1) % 2].wait()
    process(NKG - 1)

    scale = jnp.float32(1.0 / NUM_WORDS)

    def scale_body(r, carry):
        for h in range(2):
            sl = pl.ds(h * HALF, HALF)
            acc_v[r, sl] = acc_v[r, sl] * scale
        return carry

    lax.fori_loop(0, B_PER_W, scale_body, 0)
    pltpu.sync_copy(acc_v, out_hbm.at[pl.ds(obase, B_PER_W)])


_DT_C = 2048
_DT_G = (VOCAB_ + _DT_C - 1) // _DT_C
_OUT_ROWS = _DT_G * (_DT_C // 4)
PSEUDO_V = _OUT_ROWS * 128 // EMB


def _detile_body(tT_ref, out_ref):
    blk = tT_ref[...]
    out_ref[...] = jnp.concatenate(
        [blk[:, q * 512:(q + 1) * 512].T for q in range(4)], axis=1)


_detile = pl.pallas_call(
    _detile_body,
    grid=(_DT_G,),
    in_specs=[pl.BlockSpec((EMB, _DT_C), lambda c: (0, c))],
    out_specs=pl.BlockSpec((_DT_C // 4, 128), lambda c: (c, 0)),
    out_shape=jax.ShapeDtypeStruct((_OUT_ROWS, 128), jnp.float32),
)


def _headsT_body(p_ref, wmu_ref, bmu_ref, wsig_ref, bsig_ref,
                 mT_ref, lT_ref):
    pooled = p_ref[...]
    mT_ref[...] = lax.dot_general(
        wmu_ref[...], pooled, (((0,), (1,)), ((), ())),
        preferred_element_type=jnp.float32) + bmu_ref[...]
    lT_ref[...] = lax.dot_general(
        wsig_ref[...], pooled, (((0,), (1,)), ((), ())),
        preferred_element_type=jnp.float32) + bsig_ref[...]


_headsT = pl.pallas_call(
    _headsT_body,
    out_shape=(
        jax.ShapeDtypeStruct((LAT, BATCH), jnp.float32),
        jax.ShapeDtypeStruct((LAT, BATCH), jnp.float32),
    ),
)


def kernel(w, table, W_mu, b_mu, W_sig, b_sig):
    table_lin = _detile(table.T).reshape(PSEUDO_V, EMB)
    pooled = _sc_pool(w.T.astype(jnp.int32), table_lin)
    meanT, logstdT = _headsT(
        pooled, W_mu, b_mu.reshape(LAT, 1), W_sig, b_sig.reshape(LAT, 1))
    return (meanT.T, logstdT.T)


# speedup vs baseline: 14.4987x; 1.5642x over previous
import functools

import jax
import jax.numpy as jnp
from jax import lax
from jax.experimental import pallas as pl
from jax.experimental.pallas import tpu as pltpu
from jax.experimental.pallas import tpu_sc as plsc

BATCH = 4096
VOCAB_ = 1000000
NUM_WORDS = 50
EMB = 32
LAT = 64
HALF = 16

NC = 2
NS = 16
NW = NC * NS
B_PER_W = BATCH // NW
CHUNK = 16
NCHUNK = B_PER_W // CHUNK
ROWS = CHUNK * NUM_WORDS

_mesh = plsc.VectorSubcoreMesh(core_axis_name="c", subcore_axis_name="s")


KG = 10
NKG = NUM_WORDS // KG

_DT_C = 8192
_DT_Q = _DT_C // 4
_DT_SH = _DT_Q.bit_length() - 1
_DT_G = (VOCAB_ + _DT_C - 1) // _DT_C
_OUT_ROWS = _DT_G * (_DT_C // 4)
PSEUDO_V = _OUT_ROWS * 128 // EMB


@functools.partial(
    pl.kernel,
    mesh=_mesh,
    compiler_params=pltpu.CompilerParams(use_tc_tiling_on_sc=False),
    out_type=jax.ShapeDtypeStruct((BATCH, EMB), jnp.float32),
    scratch_types=[
        pltpu.VMEM((NUM_WORDS * B_PER_W,), jnp.int32),
        pltpu.VMEM((KG * B_PER_W, EMB), jnp.float32),
        pltpu.VMEM((KG * B_PER_W, EMB), jnp.float32),
        pltpu.VMEM((B_PER_W, EMB), jnp.float32),
        pltpu.SemaphoreType.DMA,
        pltpu.SemaphoreType.DMA,
        pltpu.SemaphoreType.DMA,
    ],
)
def _sc_pool(wT_hbm, table_hbm, out_hbm, idxk_v, buf0, buf1, acc_v,
             sem0, sem1, sem_fill):
    wid = lax.axis_index("s") * NC + lax.axis_index("c")
    obase = wid * B_PER_W

    fills = []
    for k in range(NUM_WORDS):
        fills.append(pltpu.async_copy(
            wT_hbm.at[k, pl.ds(obase, B_PER_W)],
            idxk_v.at[pl.ds(k * B_PER_W, B_PER_W)],
            sem_fill))
    for f in fills:
        f.wait()

    def xform_body(k, carry):
        sl = pl.ds(k * HALF, HALF)
        i = idxk_v[sl]
        v = jnp.bitwise_and(i, _DT_C - 1)
        idxk_v[sl] = ((i - v)
                      + jnp.left_shift(jnp.bitwise_and(v, _DT_Q - 1), 2)
                      + jnp.right_shift(v, _DT_SH))
        return carry

    lax.fori_loop(0, NUM_WORDS * B_PER_W // HALF, xform_body, 0)

    bufs = (buf0, buf1)
    sems = (sem0, sem1)
    handles = [None, None]

    def start(g):
        idx_sl = idxk_v.at[pl.ds(g * KG * B_PER_W, KG * B_PER_W)]
        handles[g % 2] = pltpu.async_copy(
            table_hbm.at[idx_sl], bufs[g % 2], sems[g % 2])

    def process(g):
        buf = bufs[g % 2]
        first = g == 0

        def row_body(r, carry):
            for h in range(2):
                sl = pl.ds(h * HALF, HALF)
                b = [buf[j * B_PER_W + r, sl] for j in range(KG)]
                s = (((b[0] + b[1]) + (b[2] + b[3]))
                     + ((b[4] + b[5]) + (b[6] + b[7]))) + (b[8] + b[9])
                if first:
                    acc_v[r, sl] = s
                else:
                    acc_v[r, sl] = acc_v[r, sl] + s
            return carry

        lax.fori_loop(0, B_PER_W, row_body, 0)

    start(0)
    for g in range(1, NKG):
        start(g)
        handles[(g - 1) % 2].wait()
        process(g - 1)
    handles[(NKG - 1) % 2].wait()
    process(NKG - 1)

    scale = jnp.float32(1.0 / NUM_WORDS)

    def scale_body(r, carry):
        for h in range(2):
            sl = pl.ds(h * HALF, HALF)
            acc_v[r, sl] = acc_v[r, sl] * scale
        return carry

    lax.fori_loop(0, B_PER_W, scale_body, 0)
    pltpu.sync_copy(acc_v, out_hbm.at[pl.ds(obase, B_PER_W)])


def _detile_body(tT_ref, out_ref):
    blk = tT_ref[...]
    out_ref[...] = jnp.concatenate(
        [blk[:, q * _DT_Q:(q + 1) * _DT_Q].T for q in range(4)], axis=1)


_detile = pl.pallas_call(
    _detile_body,
    grid=(_DT_G,),
    in_specs=[pl.BlockSpec((EMB, _DT_C), lambda c: (0, c))],
    out_specs=pl.BlockSpec((_DT_C // 4, 128), lambda c: (c, 0)),
    out_shape=jax.ShapeDtypeStruct((_OUT_ROWS, 128), jnp.float32),
)


def _headsT_body(p_ref, wmu_ref, bmu_ref, wsig_ref, bsig_ref,
                 mT_ref, lT_ref):
    pooled = p_ref[...]
    mT_ref[...] = lax.dot_general(
        wmu_ref[...], pooled, (((0,), (1,)), ((), ())),
        preferred_element_type=jnp.float32) + bmu_ref[...]
    lT_ref[...] = lax.dot_general(
        wsig_ref[...], pooled, (((0,), (1,)), ((), ())),
        preferred_element_type=jnp.float32) + bsig_ref[...]


_headsT = pl.pallas_call(
    _headsT_body,
    out_shape=(
        jax.ShapeDtypeStruct((LAT, BATCH), jnp.float32),
        jax.ShapeDtypeStruct((LAT, BATCH), jnp.float32),
    ),
)


def kernel(w, table, W_mu, b_mu, W_sig, b_sig):
    table_lin = _detile(table.T).reshape(PSEUDO_V, EMB)
    pooled = _sc_pool(w.T.astype(jnp.int32), table_lin)
    meanT, logstdT = _headsT(
        pooled, W_mu, b_mu.reshape(LAT, 1), W_sig, b_sig.reshape(LAT, 1))
    return (meanT.T, logstdT.T)


# speedup vs baseline: 14.6441x; 1.0100x over previous
import functools

import jax
import jax.numpy as jnp
from jax import lax
from jax.experimental import pallas as pl
from jax.experimental.pallas import tpu as pltpu
from jax.experimental.pallas import tpu_sc as plsc

BATCH = 4096
VOCAB_ = 1000000
NUM_WORDS = 50
EMB = 32
LAT = 64
HALF = 16

NC = 2
NS = 16
NW = NC * NS
B_PER_W = BATCH // NW
CHUNK = 16
NCHUNK = B_PER_W // CHUNK
ROWS = CHUNK * NUM_WORDS

_mesh = plsc.VectorSubcoreMesh(core_axis_name="c", subcore_axis_name="s")


KG = 10
NKG = NUM_WORDS // KG

_DT_C = 16384
_DT_Q = _DT_C // 4
_DT_SH = _DT_Q.bit_length() - 1
_DT_G = (VOCAB_ + _DT_C - 1) // _DT_C
_OUT_ROWS = _DT_G * (_DT_C // 4)
PSEUDO_V = _OUT_ROWS * 128 // EMB


@functools.partial(
    pl.kernel,
    mesh=_mesh,
    compiler_params=pltpu.CompilerParams(use_tc_tiling_on_sc=False),
    out_type=jax.ShapeDtypeStruct((BATCH, EMB), jnp.float32),
    scratch_types=[
        pltpu.VMEM((NUM_WORDS * B_PER_W,), jnp.int32),
        pltpu.VMEM((KG * B_PER_W, EMB), jnp.float32),
        pltpu.VMEM((KG * B_PER_W, EMB), jnp.float32),
        pltpu.VMEM((B_PER_W, EMB), jnp.float32),
        pltpu.SemaphoreType.DMA,
        pltpu.SemaphoreType.DMA,
        pltpu.SemaphoreType.DMA,
    ],
)
def _sc_pool(wT_hbm, table_hbm, out_hbm, idxk_v, buf0, buf1, acc_v,
             sem0, sem1, sem_fill):
    wid = lax.axis_index("s") * NC + lax.axis_index("c")
    obase = wid * B_PER_W

    fills = []
    for k in range(NUM_WORDS):
        fills.append(pltpu.async_copy(
            wT_hbm.at[k, pl.ds(obase, B_PER_W)],
            idxk_v.at[pl.ds(k * B_PER_W, B_PER_W)],
            sem_fill))
    for f in fills:
        f.wait()

    def xform_body(k, carry):
        sl = pl.ds(k * HALF, HALF)
        i = idxk_v[sl]
        v = jnp.bitwise_and(i, _DT_C - 1)
        idxk_v[sl] = ((i - v)
                      + jnp.left_shift(jnp.bitwise_and(v, _DT_Q - 1), 2)
                      + jnp.right_shift(v, _DT_SH))
        return carry

    lax.fori_loop(0, NUM_WORDS * B_PER_W // HALF, xform_body, 0)

    bufs = (buf0, buf1)
    sems = (sem0, sem1)
    handles = [None, None]

    def start(g):
        idx_sl = idxk_v.at[pl.ds(g * KG * B_PER_W, KG * B_PER_W)]
        handles[g % 2] = pltpu.async_copy(
            table_hbm.at[idx_sl], bufs[g % 2], sems[g % 2])

    def process(g):
        buf = bufs[g % 2]
        first = g == 0

        def row_body(r, carry):
            for h in range(2):
                sl = pl.ds(h * HALF, HALF)
                b = [buf[j * B_PER_W + r, sl] for j in range(KG)]
                s = (((b[0] + b[1]) + (b[2] + b[3]))
                     + ((b[4] + b[5]) + (b[6] + b[7]))) + (b[8] + b[9])
                if first:
                    acc_v[r, sl] = s
                else:
                    acc_v[r, sl] = acc_v[r, sl] + s
            return carry

        lax.fori_loop(0, B_PER_W, row_body, 0)

    start(0)
    for g in range(1, NKG):
        start(g)
        handles[(g - 1) % 2].wait()
        process(g - 1)
    handles[(NKG - 1) % 2].wait()
    process(NKG - 1)

    scale = jnp.float32(1.0 / NUM_WORDS)

    def scale_body(r, carry):
        for h in range(2):
            sl = pl.ds(h * HALF, HALF)
            acc_v[r, sl] = acc_v[r, sl] * scale
        return carry

    lax.fori_loop(0, B_PER_W, scale_body, 0)
    pltpu.sync_copy(acc_v, out_hbm.at[pl.ds(obase, B_PER_W)])


def _detile_body(tT_ref, out_ref):
    blk = tT_ref[...]
    out_ref[...] = jnp.concatenate(
        [blk[:, q * _DT_Q:(q + 1) * _DT_Q].T for q in range(4)], axis=1)


_detile = pl.pallas_call(
    _detile_body,
    grid=(_DT_G,),
    in_specs=[pl.BlockSpec((EMB, _DT_C), lambda c: (0, c))],
    out_specs=pl.BlockSpec((_DT_C // 4, 128), lambda c: (c, 0)),
    out_shape=jax.ShapeDtypeStruct((_OUT_ROWS, 128), jnp.float32),
)


def _headsT_body(p_ref, wmu_ref, bmu_ref, wsig_ref, bsig_ref,
                 mT_ref, lT_ref):
    pooled = p_ref[...]
    mT_ref[...] = lax.dot_general(
        wmu_ref[...], pooled, (((0,), (1,)), ((), ())),
        preferred_element_type=jnp.float32) + bmu_ref[...]
    lT_ref[...] = lax.dot_general(
        wsig_ref[...], pooled, (((0,), (1,)), ((), ())),
        preferred_element_type=jnp.float32) + bsig_ref[...]


_headsT = pl.pallas_call(
    _headsT_body,
    out_shape=(
        jax.ShapeDtypeStruct((LAT, BATCH), jnp.float32),
        jax.ShapeDtypeStruct((LAT, BATCH), jnp.float32),
    ),
)


def kernel(w, table, W_mu, b_mu, W_sig, b_sig):
    table_lin = _detile(table.T).reshape(PSEUDO_V, EMB)
    pooled = _sc_pool(w.T.astype(jnp.int32), table_lin)
    meanT, logstdT = _headsT(
        pooled, W_mu, b_mu.reshape(LAT, 1), W_sig, b_sig.reshape(LAT, 1))
    return (meanT.T, logstdT.T)
